# Initial kernel scaffold; baseline (speedup 1.0000x reference)
import jax
import jax.numpy as jnp
from jax.experimental import pallas as pl


def kernel(inputs, W):
    raise NotImplementedError("write your pallas kernel here")


# speedup vs baseline: 9.5075x; 9.5075x over previous
import functools

import jax
import jax.numpy as jnp
from jax.experimental import pallas as pl

_IN_CH = 2048
_OUT_CH = 4096
_K = 50
_DECAY = 0.95
_BATCH = 64

_NBLK = 8
_BLK = _OUT_CH // _NBLK


def _matmul_body(x_ref, w_ref, out_ref):
    out_ref[...] = jax.lax.dot_general(
        x_ref[...], w_ref[...],
        dimension_numbers=(((1,), (1,)), ((), ())),
        preferred_element_type=jnp.float32,
    )


def _filter_body(enc_ref, out_ref):
    rows = jax.lax.broadcasted_iota(jnp.int32, (_NBLK, _BLK), 0)
    cols = jax.lax.broadcasted_iota(jnp.int32, (_NBLK, _BLK), 1)
    flat = rows * _BLK + cols

    def step(i, inh):
        e = enc_ref[i]
        refr = jnp.abs(e) * (1.0 - inh)

        def extract(_, carry):
            r, m = carry
            top = jnp.max(r)
            idx = jnp.min(jnp.where(r == top, flat, _OUT_CH))
            hit = flat == idx
            m = jnp.where(hit, 1.0, m)
            r = jnp.where(hit, -jnp.inf, r)
            return r, m

        _, mask = jax.lax.fori_loop(
            0, _K, extract, (refr, jnp.zeros_like(refr)))
        out_ref[i] = e * mask
        return inh * _DECAY + mask

    jax.lax.fori_loop(0, _BATCH, step, jnp.zeros((_NBLK, _BLK), jnp.float32))


@jax.jit
def kernel(inputs, W):
    x = inputs.reshape(_BATCH, _IN_CH)
    enc = pl.pallas_call(
        _matmul_body,
        grid=(_NBLK,),
        in_specs=[
            pl.BlockSpec((_BATCH, _IN_CH), lambda j: (0, 0)),
            pl.BlockSpec((_BLK, _IN_CH), lambda j: (j, 0)),
        ],
        out_specs=pl.BlockSpec((_BATCH, _BLK), lambda j: (0, j)),
        out_shape=jax.ShapeDtypeStruct((_BATCH, _OUT_CH), jnp.float32),
    )(x, W)

    enc3 = enc.reshape(_BATCH, _NBLK, _BLK)
    out3 = pl.pallas_call(
        _filter_body,
        out_shape=jax.ShapeDtypeStruct((_BATCH, _NBLK, _BLK), jnp.float32),
    )(enc3)
    return out3.reshape(_BATCH, _OUT_CH)


# speedup vs baseline: 51.7955x; 5.4479x over previous
import functools

import jax
import jax.numpy as jnp
from jax import lax
from jax.experimental import pallas as pl
from jax.experimental.pallas import tpu as pltpu
from jax.experimental.pallas import tpu_sc as plsc

_IN_CH = 2048
_OUT_CH = 4096
_K = 50
_DECAY = 0.95
_BATCH = 64

_NBLK = 8
_BLK = _OUT_CH // _NBLK

_NT = 16
_CPT = _OUT_CH // _NT
_VPT = _CPT // 16


def _matmul_body(x_ref, w_ref, out_ref):
    out_ref[...] = jax.lax.dot_general(
        x_ref[...], w_ref[...],
        dimension_numbers=(((1,), (1,)), ((), ())),
        preferred_element_type=jnp.float32,
    )


def _merge16(ka, va, kb, vb):
    kbr = lax.rev(kb, (0,))
    vbr = lax.rev(vb, (0,))
    sel = (ka > kbr) | ((ka == kbr) & (va < vbr))
    k = jnp.where(sel, ka, kbr)
    v = jnp.where(sel, va, vbr)
    return plsc.sort_key_val(k, v, descending=True)


def _sc_filter(enc_t):
    mesh = plsc.VectorSubcoreMesh(
        core_axis_name="c", subcore_axis_name="s", num_cores=1)

    @functools.partial(
        pl.kernel,
        mesh=mesh,
        out_type=jax.ShapeDtypeStruct((_NT, _BATCH, _CPT), jnp.float32),
        compiler_params=pltpu.CompilerParams(
            needs_layout_passes=False, use_tc_tiling_on_sc=False),
        scratch_types=[
            pltpu.VMEM((_BATCH, _CPT), jnp.float32),
            pltpu.VMEM((_BATCH, _CPT), jnp.float32),
            pltpu.VMEM((_CPT,), jnp.float32),
            pltpu.VMEM((_CPT,), jnp.float32),
            pltpu.VMEM((_CPT,), jnp.float32),
            pltpu.VMEM((16,), jnp.float32),
            pltpu.VMEM((16,), jnp.int32),
            pltpu.VMEM((_NT, 16), jnp.float32),
            pltpu.VMEM((_NT, 16), jnp.int32),
            pltpu.VMEM_SHARED((_NT, 16), jnp.float32),
            pltpu.VMEM_SHARED((_NT, 16), jnp.int32),
        ],
    )
    def filt(enc_hbm, out_hbm, enc_v, out_v, refr_v, fired_v, inh_v,
             pub_k, pub_v, allk_v, allv_v, shk, shv):
        sid = lax.axis_index("s")
        pltpu.sync_copy(enc_hbm.at[sid], enc_v)

        zeros16 = jnp.zeros((16,), jnp.float32)
        ones16 = jnp.ones((16,), jnp.float32)
        ninf16 = jnp.full((16,), -jnp.inf, jnp.float32)
        lane = lax.iota(jnp.int32, 16)
        base = sid * _CPT

        for j in range(_VPT):
            inh_v[pl.ds(j * 16, 16)] = zeros16

        def step(i, carry):
            for j in range(_VPT):
                e = enc_v[i, pl.ds(j * 16, 16)]
                inh = inh_v[pl.ds(j * 16, 16)]
                refr_v[pl.ds(j * 16, 16)] = jnp.abs(e) * (1.0 - inh)
                fired_v[pl.ds(j * 16, 16)] = zeros16

            for take in (16, 16, 16, _K - 48):
                runs = []
                for j in range(_VPT):
                    kk = refr_v[pl.ds(j * 16, 16)]
                    vv = lane + (base + j * 16)
                    runs.append(plsc.sort_key_val(kk, vv, descending=True))
                while len(runs) > 1:
                    runs = [_merge16(*runs[t], *runs[t + 1])
                            for t in range(0, len(runs), 2)]
                lk, lv = runs[0]
                pub_k[...] = lk
                pub_v[...] = lv
                pltpu.sync_copy(pub_k, shk.at[sid])
                pltpu.sync_copy(pub_v, shv.at[sid])
                plsc.subcore_barrier()
                pltpu.sync_copy(shk, allk_v)
                pltpu.sync_copy(shv, allv_v)
                plsc.subcore_barrier()
                tops = [(allk_v[t], allv_v[t]) for t in range(_NT)]
                while len(tops) > 1:
                    tops = [_merge16(*tops[t], *tops[t + 1])
                            for t in range(0, len(tops), 2)]
                gk, gv = tops[0]
                local = gv - base
                own = (lane < take) & (local >= 0) & (local < _CPT)
                local_c = jnp.minimum(jnp.maximum(local, 0), _CPT - 1)
                plsc.store_scatter(fired_v, [local_c], ones16, mask=own)
                plsc.store_scatter(refr_v, [local_c], ninf16, mask=own)

            for j in range(_VPT):
                f = fired_v[pl.ds(j * 16, 16)]
                e = enc_v[i, pl.ds(j * 16, 16)]
                out_v[i, pl.ds(j * 16, 16)] = e * f
                inh = inh_v[pl.ds(j * 16, 16)]
                inh_v[pl.ds(j * 16, 16)] = inh * _DECAY + f
            return carry

        lax.fori_loop(0, _BATCH, step, 0)
        pltpu.sync_copy(out_v, out_hbm.at[sid])

    return filt(enc_t)


@jax.jit
def kernel(inputs, W):
    x = inputs.reshape(_BATCH, _IN_CH)
    enc = pl.pallas_call(
        _matmul_body,
        grid=(_NBLK,),
        in_specs=[
            pl.BlockSpec((_BATCH, _IN_CH), lambda j: (0, 0)),
            pl.BlockSpec((_BLK, _IN_CH), lambda j: (j, 0)),
        ],
        out_specs=pl.BlockSpec((_BATCH, _BLK), lambda j: (0, j)),
        out_shape=jax.ShapeDtypeStruct((_BATCH, _OUT_CH), jnp.float32),
    )(x, W)

    enc_t = enc.reshape(_BATCH, _NT, _CPT).transpose(1, 0, 2)
    out_t = _sc_filter(enc_t)
    return out_t.transpose(1, 0, 2).reshape(_BATCH, _OUT_CH)


# speedup vs baseline: 106.8554x; 2.0630x over previous
import functools

import jax
import jax.numpy as jnp
from jax import lax
from jax.experimental import pallas as pl
from jax.experimental.pallas import tpu as pltpu
from jax.experimental.pallas import tpu_sc as plsc

_IN_CH = 2048
_OUT_CH = 4096
_K = 50
_DECAY = 0.95
_BATCH = 64

_NBLK = 8
_BLK = _OUT_CH // _NBLK

_NT = 16
_CPT = _OUT_CH // _NT
_VPT = _CPT // 16


def _matmul_body(x_ref, w_ref, out_ref):
    out_ref[...] = jax.lax.dot_general(
        x_ref[...], w_ref[...],
        dimension_numbers=(((1,), (1,)), ((), ())),
        preferred_element_type=jnp.float32,
    )


def _kv_sort(kv):
    return plsc.sort_key_val(kv[0], kv[1], descending=True)


def _rev(kv):
    return (lax.rev(kv[0], (0,)), lax.rev(kv[1], (0,)))


def _exchange(a, b):
    pred = (a[0] > b[0]) | ((a[0] == b[0]) & (a[1] < b[1]))
    hi = (jnp.where(pred, a[0], b[0]), jnp.where(pred, a[1], b[1]))
    lo = (jnp.where(pred, b[0], a[0]), jnp.where(pred, b[1], a[1]))
    return hi, lo


def _take_hi(a, b):
    pred = (a[0] > b[0]) | ((a[0] == b[0]) & (a[1] < b[1]))
    return (jnp.where(pred, a[0], b[0]), jnp.where(pred, a[1], b[1]))


def _merge_16_16_full(a, b):
    hi, lo = _exchange(a, _rev(b))
    return [_kv_sort(hi), _kv_sort(lo)]


def _merge_32_32_full(a, b):
    h0, l0 = _exchange(a[0], _rev(b[1]))
    h1, l1 = _exchange(a[1], _rev(b[0]))
    hh, hl = _exchange(h0, h1)
    lh, ll = _exchange(l0, l1)
    return [_kv_sort(hh), _kv_sort(hl), _kv_sort(lh), _kv_sort(ll)]


def _merge_64_64_top(a, b):
    t = [_take_hi(a[i], _rev(b[3 - i])) for i in range(4)]
    h0, l0 = _exchange(t[0], t[2])
    h1, l1 = _exchange(t[1], t[3])
    hh, hl = _exchange(h0, h1)
    lh, ll = _exchange(l0, l1)
    return [_kv_sort(hh), _kv_sort(hl), _kv_sort(lh), _kv_sort(ll)]


def _local_top64(leaves):
    s32 = [_merge_16_16_full(leaves[t], leaves[t + 1]) for t in range(0, 16, 2)]
    s64 = [_merge_32_32_full(s32[t], s32[t + 1]) for t in range(0, 8, 2)]
    t64 = [_merge_64_64_top(s64[t], s64[t + 1]) for t in range(0, 4, 2)]
    return _merge_64_64_top(t64[0], t64[1])


def _sc_filter(enc_t):
    mesh = plsc.VectorSubcoreMesh(
        core_axis_name="c", subcore_axis_name="s", num_cores=1)

    @functools.partial(
        pl.kernel,
        mesh=mesh,
        out_type=jax.ShapeDtypeStruct((_NT, _BATCH, _CPT), jnp.float32),
        compiler_params=pltpu.CompilerParams(
            needs_layout_passes=False, use_tc_tiling_on_sc=False),
        scratch_types=[
            pltpu.VMEM((_BATCH, _CPT), jnp.float32),
            pltpu.VMEM((_BATCH, _CPT), jnp.float32),
            pltpu.VMEM((_CPT,), jnp.float32),
            pltpu.VMEM((_CPT,), jnp.float32),
            pltpu.VMEM((128,), jnp.float32),
            pltpu.VMEM((_NT, 128), jnp.float32),
            pltpu.VMEM_SHARED((_NT, 128), jnp.float32),
        ],
    )
    def filt(enc_hbm, out_hbm, enc_v, out_v, fired_v, inh_v, pub, allp, shp):
        sid = lax.axis_index("s")
        pltpu.sync_copy(enc_hbm.at[sid], enc_v)

        zeros16 = jnp.zeros((16,), jnp.float32)
        ones16 = jnp.ones((16,), jnp.float32)
        lane = lax.iota(jnp.int32, 16)
        base = sid * _CPT

        for j in range(_VPT):
            inh_v[pl.ds(j * 16, 16)] = zeros16

        def step(i, carry):
            leaves = []
            for j in range(_VPT):
                e = enc_v[i, pl.ds(j * 16, 16)]
                inh = inh_v[pl.ds(j * 16, 16)]
                kk = jnp.abs(e) * (1.0 - inh)
                vv = lane + (base + j * 16)
                leaves.append(_kv_sort((kk, vv)))
                fired_v[pl.ds(j * 16, 16)] = zeros16

            lk = _local_top64(leaves)
            for j in range(4):
                pub[pl.ds(j * 16, 16)] = lk[j][0]
                pub[pl.ds(64 + j * 16, 16)] = plsc.bitcast(lk[j][1], jnp.float32)
            pltpu.sync_copy(pub, shp.at[sid])
            plsc.subcore_barrier()
            pltpu.sync_copy(shp, allp)
            plsc.subcore_barrier()

            tops = []
            for t in range(_NT):
                tops.append([
                    (allp[t, pl.ds(j * 16, 16)],
                     plsc.bitcast(allp[t, pl.ds(64 + j * 16, 16)], jnp.int32))
                    for j in range(4)
                ])
            while len(tops) > 1:
                tops = [_merge_64_64_top(tops[t], tops[t + 1])
                        for t in range(0, len(tops), 2)]
            g = tops[0]

            for j, take in enumerate((16, 16, 16, _K - 48)):
                local = g[j][1] - base
                own = (lane < take) & (local >= 0) & (local < _CPT)
                local_c = jnp.minimum(jnp.maximum(local, 0), _CPT - 1)
                plsc.store_scatter(fired_v, [local_c], ones16, mask=own)

            for j in range(_VPT):
                f = fired_v[pl.ds(j * 16, 16)]
                e = enc_v[i, pl.ds(j * 16, 16)]
                out_v[i, pl.ds(j * 16, 16)] = e * f
                inh = inh_v[pl.ds(j * 16, 16)]
                inh_v[pl.ds(j * 16, 16)] = inh * _DECAY + f
            return carry

        lax.fori_loop(0, _BATCH, step, 0)
        pltpu.sync_copy(out_v, out_hbm.at[sid])

    return filt(enc_t)


@jax.jit
def kernel(inputs, W):
    x = inputs.reshape(_BATCH, _IN_CH)
    enc = pl.pallas_call(
        _matmul_body,
        grid=(_NBLK,),
        in_specs=[
            pl.BlockSpec((_BATCH, _IN_CH), lambda j: (0, 0)),
            pl.BlockSpec((_BLK, _IN_CH), lambda j: (j, 0)),
        ],
        out_specs=pl.BlockSpec((_BATCH, _BLK), lambda j: (0, j)),
        out_shape=jax.ShapeDtypeStruct((_BATCH, _OUT_CH), jnp.float32),
    )(x, W)

    enc_t = enc.reshape(_BATCH, _NT, _CPT).transpose(1, 0, 2)
    out_t = _sc_filter(enc_t)
    return out_t.transpose(1, 0, 2).reshape(_BATCH, _OUT_CH)
